# baseline (speedup 1.0000x reference)
import numpy as np

import jax
import jax.numpy as jnp
from jax.experimental import pallas as pl
from jax.experimental.pallas import tpu as pltpu

K_BALL = 16
RADIUS2 = 0.2 * 0.2
TILE = 512


def _body(i_ref, j_ref, pc_ref, mask_ref, pct_ref, maskt_ref, tri_ref,
          tri_lo_ref, out_ref, cnt_ref):
    b = pl.program_id(0)
    p = pl.program_id(1)
    ti = i_ref[p]
    tj = j_ref[p]
    diag = ti == tj

    @pl.when(p == 0)
    def _reset():
        cnt_ref[...] = jnp.zeros_like(cnt_ref)

    @pl.when((b == 0) & (p == 0))
    def _init():
        out_ref[...] = jnp.zeros_like(out_ref)

    pcb = pc_ref[0]
    pct = pct_ref[0]
    maskb = mask_ref[0].astype(jnp.bfloat16)
    maskt = maskt_ref[0].astype(jnp.bfloat16)
    tri = tri_ref[...]
    tri_lo = tri_lo_ref[...]

    cnt_i = cnt_ref[pl.ds(ti * TILE, TILE), :]
    cnt_j = cnt_ref[pl.ds(tj * TILE, TILE), :]
    cnt_j_t = jnp.transpose(cnt_j)
    mf = jnp.where(diag, 0.0, 1.0).astype(jnp.float32)
    mb = mf.astype(jnp.bfloat16)

    TS = 128
    rowcarry = jnp.zeros((TILE, 1), jnp.float32)
    part = jnp.zeros((), jnp.float32)
    colsums = []
    for c0 in range(0, TILE, TS):
        d2 = jnp.zeros((TILE, TS), jnp.float32)
        for c in range(3):
            diff = pcb[:, c:c + 1] - pct[c:c + 1, c0:c0 + TS]
            d2 = d2 + diff * diff
        within = d2 < RADIUS2
        wbf = within.astype(jnp.bfloat16)

        l1 = jnp.zeros((TILE, TS), jnp.bfloat16)
        for c in range(16):
            l1 = l1 + jnp.abs(maskb[:, c:c + 1] - maskt[c:c + 1, c0:c0 + TS])

        excl = jax.lax.dot_general(
            wbf, tri[c0:c0 + TS, c0:c0 + TS], (((1,), (0,)), ((), ())),
            preferred_element_type=jnp.float32)
        base = cnt_i + rowcarry
        sel = within & (base + excl < K_BALL)
        rowcarry = rowcarry + excl[:, -1:] + within[:, -1:].astype(jnp.float32)

        excl_m = jax.lax.dot_general(
            tri_lo, wbf, (((1,), (0,)), ((), ())),
            preferred_element_type=jnp.float32)
        sel_m = within & (cnt_j_t[:, c0:c0 + TS] + excl_m < K_BALL)
        colsums.append(excl_m[-1:, :] + within[-1:, :].astype(jnp.float32))

        w = sel.astype(jnp.bfloat16) + sel_m.astype(jnp.bfloat16) * mb
        part = part + jnp.sum((w * l1).astype(jnp.float32))

    colsum = jnp.concatenate(colsums, axis=1)
    cnt_ref[pl.ds(tj * TILE, TILE), :] = cnt_j + jnp.transpose(colsum) * mf
    cnt_ref[pl.ds(ti * TILE, TILE), :] = cnt_i + rowcarry
    out_ref[...] += part.reshape(1, 1)


def _run(pc, mask):
    B, N, _ = pc.shape
    nt = N // TILE
    pct = jnp.transpose(pc, (0, 2, 1))
    maskt = jnp.transpose(mask, (0, 2, 1))
    tri = (jnp.arange(TILE)[:, None] < jnp.arange(TILE)[None, :]
           ).astype(jnp.bfloat16)
    tri_lo = jnp.transpose(tri)
    pairs = sorted(((i, j) for i in range(nt) for j in range(i, nt)),
                   key=lambda ij: (ij[0] + ij[1], ij[0]))
    i_arr = np.array([ij[0] for ij in pairs], dtype=np.int32)
    j_arr = np.array([ij[1] for ij in pairs], dtype=np.int32)

    total = pl.pallas_call(
        _body,
        grid_spec=pltpu.PrefetchScalarGridSpec(
            num_scalar_prefetch=2,
            grid=(B, len(pairs)),
            in_specs=[
                pl.BlockSpec((1, TILE, 3),
                             lambda b, p, i, j: (b, i[p], 0)),
                pl.BlockSpec((1, TILE, 16),
                             lambda b, p, i, j: (b, i[p], 0)),
                pl.BlockSpec((1, 3, TILE),
                             lambda b, p, i, j: (b, 0, j[p])),
                pl.BlockSpec((1, 16, TILE),
                             lambda b, p, i, j: (b, 0, j[p])),
                pl.BlockSpec((TILE, TILE),
                             lambda b, p, i, j: (0, 0)),
                pl.BlockSpec((TILE, TILE),
                             lambda b, p, i, j: (0, 0)),
            ],
            out_specs=pl.BlockSpec((1, 1), lambda b, p, i, j: (0, 0)),
            scratch_shapes=[pltpu.VMEM((N, 1), jnp.float32)],
        ),
        out_shape=jax.ShapeDtypeStruct((1, 1), jnp.float32),
        compiler_params=pltpu.CompilerParams(
            dimension_semantics=("arbitrary", "arbitrary")),
    )(jnp.asarray(i_arr), jnp.asarray(j_arr), pc, mask, pct, maskt, tri,
      tri_lo)
    return total


def kernel(pc, mask):
    B, N, _ = pc.shape
    total = _run(pc, mask)
    return total[0, 0] / (B * N * K_BALL)


# speedup vs baseline: 1.1728x; 1.1728x over previous
import numpy as np

import jax
import jax.numpy as jnp
from jax.experimental import pallas as pl
from jax.experimental.pallas import tpu as pltpu

K_BALL = 16
RADIUS2 = 0.2 * 0.2
TILE = 512


def _body(i_ref, j_ref, pc_ref, mask_ref, pct_ref, maskt_ref, tri_ref,
          tri_lo_ref, out_ref, cnt_ref):
    b = pl.program_id(0)
    p = pl.program_id(1)
    ti = i_ref[p]
    tj = j_ref[p]
    diag = ti == tj

    @pl.when(p == 0)
    def _reset():
        cnt_ref[...] = jnp.zeros_like(cnt_ref)

    @pl.when((b == 0) & (p == 0))
    def _init():
        out_ref[...] = jnp.zeros_like(out_ref)

    pcb = pc_ref[0]
    pct = pct_ref[0]
    maskb = mask_ref[0].astype(jnp.bfloat16)
    maskt = maskt_ref[0].astype(jnp.bfloat16)
    tri = tri_ref[...]

    d2 = jnp.zeros((TILE, TILE), jnp.float32)
    for c in range(3):
        diff = pcb[:, c:c + 1] - pct[c:c + 1, :]
        d2 = d2 + diff * diff
    within = d2 < RADIUS2
    wbf = within.astype(jnp.bfloat16)

    l1a = jnp.zeros((TILE, TILE), jnp.bfloat16)
    l1b = jnp.zeros((TILE, TILE), jnp.bfloat16)
    for c in range(0, 16, 2):
        l1a = l1a + jnp.abs(maskb[:, c:c + 1] - maskt[c:c + 1, :])
        l1b = l1b + jnp.abs(maskb[:, c + 1:c + 2] - maskt[c + 1:c + 2, :])
    l1 = l1a + l1b

    cnt_i = cnt_ref[pl.ds(ti * TILE, TILE), :]
    cnt_j = cnt_ref[pl.ds(tj * TILE, TILE), :]

    excl = jax.lax.dot_general(
        wbf, tri, (((1,), (0,)), ((), ())),
        preferred_element_type=jnp.float32)
    sel = within & (cnt_i + excl < K_BALL)
    selw = sel.astype(jnp.bfloat16)

    excl_m = jax.lax.dot_general(
        tri_lo_ref[...], wbf, (((1,), (0,)), ((), ())),
        preferred_element_type=jnp.float32)
    sel_m = within & (jnp.transpose(cnt_j) + excl_m < K_BALL)
    colsum = excl_m[-1:, :] + within[-1:, :].astype(jnp.float32)

    mf = jnp.where(diag, 0.0, 1.0).astype(jnp.float32)
    mb = mf.astype(jnp.bfloat16)
    cnt_ref[pl.ds(tj * TILE, TILE), :] = cnt_j + jnp.transpose(colsum) * mf
    cnt_ref[pl.ds(ti * TILE, TILE), :] = (
        cnt_i + excl[:, -1:] + within[:, -1:].astype(jnp.float32))

    w = selw + sel_m.astype(jnp.bfloat16) * mb
    part = jnp.sum((w * l1).astype(jnp.float32))
    out_ref[...] += part.reshape(1, 1)


def _run(pc, mask):
    B, N, _ = pc.shape
    nt = N // TILE
    pct = jnp.transpose(pc, (0, 2, 1))
    maskt = jnp.transpose(mask, (0, 2, 1))
    tri = (jnp.arange(TILE)[:, None] < jnp.arange(TILE)[None, :]
           ).astype(jnp.bfloat16)
    tri_lo = jnp.transpose(tri)
    pairs = sorted(((i, j) for i in range(nt) for j in range(i, nt)),
                   key=lambda ij: (ij[0] + ij[1], ij[0]))
    i_arr = np.array([ij[0] for ij in pairs], dtype=np.int32)
    j_arr = np.array([ij[1] for ij in pairs], dtype=np.int32)

    total = pl.pallas_call(
        _body,
        grid_spec=pltpu.PrefetchScalarGridSpec(
            num_scalar_prefetch=2,
            grid=(B, len(pairs)),
            in_specs=[
                pl.BlockSpec((1, TILE, 3),
                             lambda b, p, i, j: (b, i[p], 0)),
                pl.BlockSpec((1, TILE, 16),
                             lambda b, p, i, j: (b, i[p], 0)),
                pl.BlockSpec((1, 3, TILE),
                             lambda b, p, i, j: (b, 0, j[p])),
                pl.BlockSpec((1, 16, TILE),
                             lambda b, p, i, j: (b, 0, j[p])),
                pl.BlockSpec((TILE, TILE),
                             lambda b, p, i, j: (0, 0)),
                pl.BlockSpec((TILE, TILE),
                             lambda b, p, i, j: (0, 0)),
            ],
            out_specs=pl.BlockSpec((1, 1), lambda b, p, i, j: (0, 0)),
            scratch_shapes=[pltpu.VMEM((N, 1), jnp.float32)],
        ),
        out_shape=jax.ShapeDtypeStruct((1, 1), jnp.float32),
        compiler_params=pltpu.CompilerParams(
            dimension_semantics=("arbitrary", "arbitrary")),
    )(jnp.asarray(i_arr), jnp.asarray(j_arr), pc, mask, pct, maskt, tri,
      tri_lo)
    return total


def kernel(pc, mask):
    B, N, _ = pc.shape
    total = _run(pc, mask)
    return total[0, 0] / (B * N * K_BALL)
